# baseline (speedup 1.0000x reference)
import functools

import jax
import jax.numpy as jnp
from jax import lax
from jax.experimental import pallas as pl
from jax.experimental.pallas import tpu as pltpu
from jax.experimental.pallas import tpu_sc as plsc

_N = 16384
_D = 32
_NC = 2
_NS = 16
_NW = _NC * _NS
_BPW = _N // _NW
_GRP = 16


def _sc_scores(u2, i2, j2, bu, bi, pT, qT):
    mesh = plsc.VectorSubcoreMesh(core_axis_name="c", subcore_axis_name="s")

    @functools.partial(
        pl.kernel,
        mesh=mesh,
        compiler_params=pltpu.CompilerParams(
            use_tc_tiling_on_sc=False, needs_layout_passes=False
        ),
        out_type=(
            jax.ShapeDtypeStruct((_NW, _BPW), jnp.float32),
            jax.ShapeDtypeStruct((_NW, _BPW), jnp.float32),
        ),
        scratch_types=[
            pltpu.VMEM((_BPW,), jnp.int32),
            pltpu.VMEM((_BPW,), jnp.int32),
            pltpu.VMEM((_BPW,), jnp.int32),
            pltpu.VMEM((_D, _BPW), jnp.float32),
            pltpu.VMEM((_D, _BPW), jnp.float32),
            pltpu.VMEM((_D, _BPW), jnp.float32),
            pltpu.VMEM((_BPW,), jnp.float32),
            pltpu.VMEM((_BPW,), jnp.float32),
            pltpu.VMEM((_BPW,), jnp.float32),
            pltpu.VMEM((_BPW,), jnp.float32),
            pltpu.VMEM((_BPW,), jnp.float32),
            pltpu.SemaphoreType.DMA,
        ],
    )
    def k(u_hbm, i_hbm, j_hbm, bu_hbm, bi_hbm, p_hbm, q_hbm,
          rui_hbm, ruj_hbm,
          u_v, i_v, j_v, pu_v, qi_v, qj_v, gbu_v, gbi_v, gbj_v,
          rui_v, ruj_v, sem):
        wid = lax.axis_index("s") * _NC + lax.axis_index("c")

        pltpu.sync_copy(u_hbm.at[wid], u_v)
        pltpu.sync_copy(i_hbm.at[wid], i_v)
        pltpu.sync_copy(j_hbm.at[wid], j_v)

        cps = [
            pltpu.async_copy(bu_hbm.at[u_v], gbu_v, sem),
            pltpu.async_copy(bi_hbm.at[i_v], gbi_v, sem),
            pltpu.async_copy(bi_hbm.at[j_v], gbj_v, sem),
        ]
        for d in range(_D):
            cps.append(pltpu.async_copy(p_hbm.at[d].at[u_v], pu_v.at[d], sem))
            cps.append(pltpu.async_copy(q_hbm.at[d].at[i_v], qi_v.at[d], sem))
            cps.append(pltpu.async_copy(q_hbm.at[d].at[j_v], qj_v.at[d], sem))
        for cp in cps:
            cp.wait()

        def body(g, carry):
            sl = pl.ds(g * _GRP, _GRP)
            acc_i = jnp.zeros((_GRP,), jnp.float32)
            acc_j = jnp.zeros((_GRP,), jnp.float32)
            for d in range(_D):
                vp = pu_v[d, sl]
                acc_i = acc_i + vp * qi_v[d, sl]
                acc_j = acc_j + vp * qj_v[d, sl]
            b_u = gbu_v[sl]
            rui_v[sl] = b_u + gbi_v[sl] + acc_i
            ruj_v[sl] = b_u + gbj_v[sl] + acc_j
            return carry

        lax.fori_loop(0, _BPW // _GRP, body, 0)

        pltpu.sync_copy(rui_v, rui_hbm.at[wid])
        pltpu.sync_copy(ruj_v, ruj_hbm.at[wid])

    return k(u2, i2, j2, bu, bi, pT, qT)


def _loss_body(m_ref, rui_ref, ruj_ref, ui_ref, uj_ref, out_ref):
    m = m_ref[0]
    rui = rui_ref[...] + m
    ruj = ruj_ref[...] + m
    r = rui - ruj
    bpr = jnp.maximum(-r, 0.0) + jnp.log1p(jnp.exp(-jnp.abs(r)))
    d1 = rui - ui_ref[...]
    a1 = jnp.abs(d1)
    s1 = jnp.where(a1 < 1.0, 0.5 * d1 * d1, a1 - 0.5)
    d2 = ruj - uj_ref[...]
    a2 = jnp.abs(d2)
    s2 = jnp.where(a2 < 1.0, 0.5 * d2 * d2, a2 - 0.5)
    out_ref[0, 0] = (0.7 * jnp.mean(bpr)
                     + 0.3 * 0.5 * (jnp.mean(s1) + jnp.mean(s2)))


def kernel(u, i, j, ui, uj, m, bu, bi, p, q):
    u2 = jnp.reshape(u.astype(jnp.int32), (_NW, _BPW))
    i2 = jnp.reshape(i.astype(jnp.int32), (_NW, _BPW))
    j2 = jnp.reshape(j.astype(jnp.int32), (_NW, _BPW))
    rui, ruj = _sc_scores(u2, i2, j2, bu, bi, p.T, q.T)
    out = pl.pallas_call(
        _loss_body,
        out_shape=jax.ShapeDtypeStruct((1, 1), jnp.float32),
        in_specs=[
            pl.BlockSpec(memory_space=pltpu.SMEM),
            pl.BlockSpec(memory_space=pltpu.VMEM),
            pl.BlockSpec(memory_space=pltpu.VMEM),
            pl.BlockSpec(memory_space=pltpu.VMEM),
            pl.BlockSpec(memory_space=pltpu.VMEM),
        ],
        out_specs=pl.BlockSpec(memory_space=pltpu.SMEM),
    )(m, rui, ruj, jnp.reshape(ui, (_NW, _BPW)), jnp.reshape(uj, (_NW, _BPW)))
    return out[0, 0]


# speedup vs baseline: 5.1611x; 5.1611x over previous
import functools

import jax
import jax.numpy as jnp
from jax import lax
from jax.experimental import pallas as pl
from jax.experimental.pallas import tpu as pltpu
from jax.experimental.pallas import tpu_sc as plsc

_N = 16384
_D = 32
_V = 1_000_000
_NC = 2
_NS = 16
_NW = _NC * _NS
_BPW = _N // _NW
_CHUNK = 128
_NCHUNK = _BPW // _CHUNK
_GRP = 16
_R = 128
_WB = 1024
_WG = -(-_V // _WB)


def _pack_body(p_ref, q_ref, bu_ref, bi_ref, o_ref):
    tp = jnp.transpose(p_ref[...], (1, 0))
    tq = jnp.transpose(q_ref[...], (1, 0))
    bu_c = bu_ref[...][:, None]
    bi_c = bi_ref[...][:, None]
    pad = jnp.zeros((_WB, 128 - 2 * _D - 2), jnp.float32)
    o_ref[...] = jnp.concatenate([tp, tq, bu_c, bi_c, pad], axis=1)


def _pack_tables(pT, qT, bu, bi):
    return pl.pallas_call(
        _pack_body,
        grid=(_WG,),
        in_specs=[
            pl.BlockSpec((_D, _WB), lambda b: (0, b)),
            pl.BlockSpec((_D, _WB), lambda b: (0, b)),
            pl.BlockSpec((_WB,), lambda b: (b,)),
            pl.BlockSpec((_WB,), lambda b: (b,)),
        ],
        out_specs=pl.BlockSpec((_WB, 128), lambda b: (b, 0)),
        out_shape=jax.ShapeDtypeStruct((_V, 128), jnp.float32),
    )(pT, qT, bu, bi)


def _sc_scores(u2, i2, j2, w_tab):
    mesh = plsc.VectorSubcoreMesh(core_axis_name="c", subcore_axis_name="s")

    @functools.partial(
        pl.kernel,
        mesh=mesh,
        compiler_params=pltpu.CompilerParams(
            use_tc_tiling_on_sc=False, needs_layout_passes=False
        ),
        out_type=(
            jax.ShapeDtypeStruct((_R, _R), jnp.float32),
            jax.ShapeDtypeStruct((_R, _R), jnp.float32),
        ),
        scratch_types=[
            pltpu.VMEM((_NCHUNK, _CHUNK), jnp.int32),
            pltpu.VMEM((_NCHUNK, _CHUNK), jnp.int32),
            pltpu.VMEM((_NCHUNK, _CHUNK), jnp.int32),
            pltpu.VMEM((2, _CHUNK, 128), jnp.float32),
            pltpu.VMEM((2, _CHUNK, 128), jnp.float32),
            pltpu.VMEM((2, _CHUNK, 128), jnp.float32),
            pltpu.VMEM((_NCHUNK, _CHUNK), jnp.float32),
            pltpu.VMEM((_NCHUNK, _CHUNK), jnp.float32),
            pltpu.SemaphoreType.DMA,
        ],
    )
    def k(u_hbm, i_hbm, j_hbm, w_hbm,
          rui_hbm, ruj_hbm,
          u_v, i_v, j_v, wu_v, wi_v, wj_v,
          rui_v, ruj_v, sem):
        wid = lax.axis_index("s") * _NC + lax.axis_index("c")
        row0 = wid * _NCHUNK

        pltpu.sync_copy(u_hbm.at[pl.ds(row0, _NCHUNK)], u_v)
        pltpu.sync_copy(i_hbm.at[pl.ds(row0, _NCHUNK)], i_v)
        pltpu.sync_copy(j_hbm.at[pl.ds(row0, _NCHUNK)], j_v)

        def fire(c):
            b = c % 2
            return (
                pltpu.async_copy(w_hbm.at[u_v.at[c]], wu_v.at[b], sem),
                pltpu.async_copy(w_hbm.at[i_v.at[c]], wi_v.at[b], sem),
                pltpu.async_copy(w_hbm.at[j_v.at[c]], wj_v.at[b], sem),
            )

        lane = lax.broadcasted_iota(jnp.int32, (_GRP,), 0)
        cps = {0: fire(0)}
        for c in range(_NCHUNK):
            if c + 1 < _NCHUNK:
                cps[c + 1] = fire(c + 1)
            for cp in cps.pop(c):
                cp.wait()
            b = c % 2
            bvec = jnp.full((_GRP,), b, jnp.int32)
            cvec = jnp.full((_GRP,), c, jnp.int32)

            def body(g, carry, bvec=bvec, cvec=cvec):
                row = g * _GRP + lane
                acc_i = jnp.zeros((_GRP,), jnp.float32)
                acc_j = jnp.zeros((_GRP,), jnp.float32)
                for dd in range(_D):
                    pcol = jnp.full((_GRP,), dd, jnp.int32)
                    qcol = jnp.full((_GRP,), _D + dd, jnp.int32)
                    pu_d = plsc.load_gather(wu_v, [bvec, row, pcol])
                    qi_d = plsc.load_gather(wi_v, [bvec, row, qcol])
                    qj_d = plsc.load_gather(wj_v, [bvec, row, qcol])
                    acc_i = acc_i + pu_d * qi_d
                    acc_j = acc_j + pu_d * qj_d
                bu_col = jnp.full((_GRP,), 2 * _D, jnp.int32)
                bi_col = jnp.full((_GRP,), 2 * _D + 1, jnp.int32)
                b_u = plsc.load_gather(wu_v, [bvec, row, bu_col])
                b_i = plsc.load_gather(wi_v, [bvec, row, bi_col])
                b_j = plsc.load_gather(wj_v, [bvec, row, bi_col])
                plsc.store_scatter(rui_v, [cvec, row], b_u + b_i + acc_i)
                plsc.store_scatter(ruj_v, [cvec, row], b_u + b_j + acc_j)
                return carry

            lax.fori_loop(0, _CHUNK // _GRP, body, 0)

        pltpu.sync_copy(rui_v, rui_hbm.at[pl.ds(row0, _NCHUNK)])
        pltpu.sync_copy(ruj_v, ruj_hbm.at[pl.ds(row0, _NCHUNK)])

    return k(u2, i2, j2, w_tab)


def _loss_body(m_ref, rui_ref, ruj_ref, ui_ref, uj_ref, out_ref):
    m = m_ref[0]
    rui = rui_ref[...] + m
    ruj = ruj_ref[...] + m
    r = rui - ruj
    bpr = jnp.maximum(-r, 0.0) + jnp.log1p(jnp.exp(-jnp.abs(r)))
    d1 = rui - ui_ref[...]
    a1 = jnp.abs(d1)
    s1 = jnp.where(a1 < 1.0, 0.5 * d1 * d1, a1 - 0.5)
    d2 = ruj - uj_ref[...]
    a2 = jnp.abs(d2)
    s2 = jnp.where(a2 < 1.0, 0.5 * d2 * d2, a2 - 0.5)
    out_ref[0, 0] = (0.7 * jnp.mean(bpr)
                     + 0.3 * 0.5 * (jnp.mean(s1) + jnp.mean(s2)))


def kernel(u, i, j, ui, uj, m, bu, bi, p, q):
    u2 = jnp.reshape(u.astype(jnp.int32), (_R, _R))
    i2 = jnp.reshape(i.astype(jnp.int32), (_R, _R))
    j2 = jnp.reshape(j.astype(jnp.int32), (_R, _R))
    w_tab = _pack_tables(p.T, q.T, bu, bi)
    rui, ruj = _sc_scores(u2, i2, j2, w_tab)
    out = pl.pallas_call(
        _loss_body,
        out_shape=jax.ShapeDtypeStruct((1, 1), jnp.float32),
        in_specs=[
            pl.BlockSpec(memory_space=pltpu.SMEM),
            pl.BlockSpec(memory_space=pltpu.VMEM),
            pl.BlockSpec(memory_space=pltpu.VMEM),
            pl.BlockSpec(memory_space=pltpu.VMEM),
            pl.BlockSpec(memory_space=pltpu.VMEM),
        ],
        out_specs=pl.BlockSpec(memory_space=pltpu.SMEM),
    )(m, rui, ruj, jnp.reshape(ui, (_R, _R)), jnp.reshape(uj, (_R, _R)))
    return out[0, 0]
